# baseline (speedup 1.0000x reference)
import functools

import jax
import jax.numpy as jnp
from jax import lax
from jax.experimental import pallas as pl
from jax.experimental.pallas import tpu as pltpu
from jax.experimental.pallas import tpu_sc as plsc

_N = 10000
_E = 320000
_EH = _E // 2
_H = 128
_NL = 3
_OUT = 10

_NC = 2
_NS = 16
_NW = _NC * _NS

_GCH = 40
_GPT = _EH // _NW
_GNCH = _GPT // _GCH

_SCH = 80
_SCHP = 40
_SPT = _EH // _NS
_SNCH = _SPT // _SCH
_NH = _N // 2
_NHP = 5008
_RPT = 312
_ZCH = 52

_BE = 2000
_BN = 2000


def _dot(a, b):
    return jnp.dot(a, b, preferred_element_type=jnp.float32)


def _mesh():
    return plsc.VectorSubcoreMesh(core_axis_name="c", subcore_axis_name="s",
                                  num_cores=_NC, num_subcores=_NS)


def _gather_body(ph, pq_hbm, idx_hbm, out_hbm, idxp_v, idxq_v,
                 rows0, rows1, rows2, g0, g1, g2, w0, w1, w2):
    c = lax.axis_index("c")
    s = lax.axis_index("s")
    w = c * _NS + s
    base = ph * _EH + w * _GPT
    obase = w * _GPT
    pltpu.sync_copy(idx_hbm.at[pl.ds(base, _GPT)], idxp_v)
    pltpu.sync_copy(idx_hbm.at[pl.ds(_E + base, _GPT)], idxq_v)
    rows = (rows0, rows1, rows2)
    gs = (g0, g1, g2)
    ws = (w0, w1, w2)

    def start_p(j, b):
        off = pl.multiple_of(j * _GCH, 8)
        pltpu.async_copy(pq_hbm.at[idxp_v.at[pl.ds(off, _GCH)]], rows[b], gs[b])

    def wait_g(b):
        pltpu.make_async_copy(pq_hbm.at[pl.ds(0, _GCH)], rows[b], gs[b]).wait()

    def wait_w(b):
        pltpu.make_async_copy(rows[b], out_hbm.at[pl.ds(0, _GCH)], ws[b]).wait()

    for b in range(3):
        start_p(b, b)

    def step(jj, b, dyn):
        bp = (b + 2) % 3
        wait_g(b)
        off = pl.multiple_of(jj * _GCH, 8)
        pltpu.async_copy(pq_hbm.at[idxq_v.at[pl.ds(off, _GCH)]], rows[b],
                         gs[b], add=True)
        wait_g(b)
        dst = pl.multiple_of(obase + jj * _GCH, 8)
        pltpu.async_copy(rows[b], out_hbm.at[pl.ds(dst, _GCH)], ws[b])
        if dyn:
            @pl.when((jj >= 1) & (jj + 2 < _GNCH))
            def _():
                wait_w(bp)
                start_p(jj + 2, bp)

    def body(i, carry):
        for b in range(3):
            step(3 * i + b, b, True)
        return carry

    nfull = _GNCH - _GNCH % 3
    lax.fori_loop(0, nfull // 3, body, 0)
    for jj in range(nfull, _GNCH):
        step(jj, jj % 3, False)
    for b in range(3):
        wait_w(b)


def _gather(pq2, gidx, ph):
    f = pl.kernel(
        functools.partial(_gather_body, ph),
        out_type=jax.ShapeDtypeStruct((_EH, _H), jnp.float32),
        mesh=_mesh(),
        scratch_types=[
            pltpu.VMEM((_GPT,), jnp.int32),
            pltpu.VMEM((_GPT,), jnp.int32),
            pltpu.VMEM((_GCH, _H), jnp.float32),
            pltpu.VMEM((_GCH, _H), jnp.float32),
            pltpu.VMEM((_GCH, _H), jnp.float32),
            pltpu.SemaphoreType.DMA,
            pltpu.SemaphoreType.DMA,
            pltpu.SemaphoreType.DMA,
            pltpu.SemaphoreType.DMA,
            pltpu.SemaphoreType.DMA,
            pltpu.SemaphoreType.DMA,
        ],
    )
    return f(pq2, gidx)


def _scatter_body(ph, ef_hbm, idx_hbm, out_hbm, idx1_v, idx2_v, rp0, rp1,
                  rp2, rf0, rf1, rf2, zbuf, agg_s, g0, g1, g2, a0, a1, a2):
    c = lax.axis_index("c")
    s = lax.axis_index("s")
    base_e = s * _SPT

    def zb(i, carry):
        for k in range(8):
            zbuf[i, pl.ds(k * 16, 16)] = jnp.zeros((16,), jnp.float32)
        return carry

    lax.fori_loop(0, _ZCH, zb, 0)
    zbase = pl.multiple_of(s * _RPT, 8)
    for r in range(6):
        pltpu.sync_copy(zbuf, agg_s.at[pl.ds(zbase + r * _ZCH, _ZCH)])

    @pl.when(s == _NS - 1)
    def _():
        pltpu.sync_copy(zbuf.at[pl.ds(0, 16)], agg_s.at[pl.ds(_NS * _RPT, 16)])

    pltpu.sync_copy(idx_hbm.at[pl.ds(ph * _EH + base_e, _SPT)], idx1_v)
    lo = c * _NH

    def rp(i, carry):
        src = pl.multiple_of(i * _SCH, 16)
        for k in range(5):
            v = idx1_v[pl.ds(src + k * 16, 16)] - lo
            ok = (v >= 0) & (v < _NH)
            idx2_v[i, pl.ds(k * 16, 16)] = jnp.where(ok, v, _NH)
        return carry

    lax.fori_loop(0, _SNCH, rp, 0)
    plsc.subcore_barrier()

    rowsp = (rp0, rp1, rp2)
    rowsf = (rf0, rf1, rf2)
    gs = (g0, g1, g2)
    ads = (a0, a1, a2)
    pbase = s * (_SPT // 2)

    def start(j, b):
        off = pl.multiple_of(pbase + j * _SCHP, 8)
        pltpu.async_copy(ef_hbm.at[pl.ds(off, _SCHP)], rowsp[b], gs[b])

    def wait_g(b):
        pltpu.make_async_copy(ef_hbm.at[pl.ds(0, _SCHP)], rowsp[b], gs[b]).wait()

    def wait_a(b):
        pltpu.make_async_copy(rowsf[b], agg_s.at[pl.ds(0, _SCH)], ads[b]).wait()

    for b in range(3):
        start(b, b)

    hi16 = jnp.uint32(0xFFFF0000)

    def step(jj, b, dyn):
        wait_g(b)
        if dyn:
            @pl.when(jj >= 3)
            def _():
                wait_a(b)
        else:
            wait_a(b)

        def cv(r, carry):
            for k in range(8):
                v = rowsp[b][r, pl.ds(k * 16, 16)]
                rowsf[b][2 * r, pl.ds(k * 16, 16)] = (
                    jax.lax.bitcast_convert_type(v & hi16, jnp.float32))
                rowsf[b][2 * r + 1, pl.ds(k * 16, 16)] = (
                    jax.lax.bitcast_convert_type(v << 16, jnp.float32))
            return carry

        lax.fori_loop(0, _SCHP, cv, 0)
        if dyn:
            @pl.when(jj + 3 < _SNCH)
            def _():
                start(jj + 3, b)
        pltpu.async_copy(rowsf[b], agg_s.at[idx2_v.at[jj]], ads[b], add=True)

    def body(i, carry):
        for b in range(3):
            step(3 * i + b, b, True)
        return carry

    nfull = _SNCH - _SNCH % 3
    lax.fori_loop(0, nfull // 3, body, 0)
    for jj in range(nfull, _SNCH):
        step(jj, jj % 3, False)
    for b in range(3):
        wait_a(b)
    plsc.subcore_barrier()
    src = pl.multiple_of(s * _RPT, 8)
    dst = pl.multiple_of(c * _NH + s * _RPT, 8)
    pltpu.sync_copy(agg_s.at[pl.ds(src, _RPT)], out_hbm.at[pl.ds(dst, _RPT)])

    @pl.when(s == _NS - 1)
    def _():
        tail = _NS * _RPT
        pltpu.sync_copy(agg_s.at[pl.ds(tail, 8)],
                        out_hbm.at[pl.ds(c * _NH + tail, 8)])


def _scatter(ef, row, ph):
    f = pl.kernel(
        functools.partial(_scatter_body, ph),
        out_type=jax.ShapeDtypeStruct((_N, _H), jnp.float32),
        mesh=_mesh(),
        scratch_types=[
            pltpu.VMEM((_SPT,), jnp.int32),
            pltpu.VMEM((_SNCH, _SCH), jnp.int32),
            pltpu.VMEM((_SCHP, _H), jnp.uint32),
            pltpu.VMEM((_SCHP, _H), jnp.uint32),
            pltpu.VMEM((_SCHP, _H), jnp.uint32),
            pltpu.VMEM((_SCH, _H), jnp.float32),
            pltpu.VMEM((_SCH, _H), jnp.float32),
            pltpu.VMEM((_SCH, _H), jnp.float32),
            pltpu.VMEM((_ZCH, _H), jnp.float32),
            pltpu.VMEM_SHARED((_NHP, _H), jnp.float32),
            pltpu.SemaphoreType.DMA,
            pltpu.SemaphoreType.DMA,
            pltpu.SemaphoreType.DMA,
            pltpu.SemaphoreType.DMA,
            pltpu.SemaphoreType.DMA,
            pltpu.SemaphoreType.DMA,
        ],
    )
    return f(ef, row)


def _pre_body(h, Wi, bi, A, B, xo, pqo):
    x = _dot(h[...], Wi[...]) + bi[...]
    xo[...] = x
    pqo[0, :, :] = _dot(x, A[...])
    pqo[1, :, :] = _dot(x, B[...])


def _pre(h, Wi, bi, A, B):
    grid = _N // _BN
    return pl.pallas_call(
        _pre_body,
        grid=(grid,),
        in_specs=[
            pl.BlockSpec((_BN, _H), lambda i: (i, 0)),
            pl.BlockSpec((_H, _H), lambda i: (0, 0)),
            pl.BlockSpec((1, _H), lambda i: (0, 0)),
            pl.BlockSpec((_H, _H), lambda i: (0, 0)),
            pl.BlockSpec((_H, _H), lambda i: (0, 0)),
        ],
        out_specs=[
            pl.BlockSpec((_BN, _H), lambda i: (i, 0)),
            pl.BlockSpec((2, _BN, _H), lambda i: (0, i, 0)),
        ],
        out_shape=[
            jax.ShapeDtypeStruct((_N, _H), jnp.float32),
            jax.ShapeDtypeStruct((2, _N, _H), jnp.float32),
        ],
    )(h, Wi, bi, A, B)


def _edge_body(ss, d, em, wd, b1, W2, b2, o):
    h1 = jnp.maximum(ss[...] + d[...] * wd[...] + b1[...], 0.0)
    h16 = h1.astype(jnp.bfloat16)
    m = jnp.maximum(_dot(h16, W2[...].astype(jnp.bfloat16)) + b2[...], 0.0)
    ef = m * em[...]
    e3 = ef.reshape(_BE // 2, 2, _H)
    ua = jax.lax.bitcast_convert_type(e3[:, 0, :], jnp.uint32)
    ub = jax.lax.bitcast_convert_type(e3[:, 1, :], jnp.uint32)
    o[...] = ((ua + 0x8000) & jnp.uint32(0xFFFF0000)) | ((ub + 0x8000) >> 16)


def _edge(g, d, em, wd, b1, W2, b2, ph):
    grid = _EH // _BE
    nb = _EH // _BE

    def eix(i):
        return (i + ph * nb, 0)

    return pl.pallas_call(
        _edge_body,
        grid=(grid,),
        in_specs=[
            pl.BlockSpec((_BE, _H), lambda i: (i, 0)),
            pl.BlockSpec((_BE, 1), eix),
            pl.BlockSpec((_BE, 1), eix),
            pl.BlockSpec((1, _H), lambda i: (0, 0)),
            pl.BlockSpec((1, _H), lambda i: (0, 0)),
            pl.BlockSpec((_H, _H), lambda i: (0, 0)),
            pl.BlockSpec((1, _H), lambda i: (0, 0)),
        ],
        out_specs=pl.BlockSpec((_BE // 2, _H), lambda i: (i, 0)),
        out_shape=jax.ShapeDtypeStruct((_EH // 2, _H), jnp.uint32),
    )(g, d, em, wd, b1, W2, b2)


def _node_body(x, a0, a1, W1a, W1b, b1, W2, b2, nm, A, B, xo, pqo):
    xx = x[...]
    agg = a0[...] + a1[...]
    t = jnp.maximum(_dot(xx, W1a[...]) + _dot(agg, W1b[...]) + b1[...], 0.0)
    out = (xx + _dot(t, W2[...]) + b2[...]) * nm[...]
    xo[...] = out
    pqo[0, :, :] = _dot(out, A[...])
    pqo[1, :, :] = _dot(out, B[...])


def _node(x, ag0, ag1, W1a, W1b, b1, W2, b2, nm, A, B):
    grid = _N // _BN
    return pl.pallas_call(
        _node_body,
        grid=(grid,),
        in_specs=[
            pl.BlockSpec((_BN, _H), lambda i: (i, 0)),
            pl.BlockSpec((_BN, _H), lambda i: (i, 0)),
            pl.BlockSpec((_BN, _H), lambda i: (i, 0)),
            pl.BlockSpec((_H, _H), lambda i: (0, 0)),
            pl.BlockSpec((_H, _H), lambda i: (0, 0)),
            pl.BlockSpec((1, _H), lambda i: (0, 0)),
            pl.BlockSpec((_H, _H), lambda i: (0, 0)),
            pl.BlockSpec((1, _H), lambda i: (0, 0)),
            pl.BlockSpec((_BN, 1), lambda i: (i, 0)),
            pl.BlockSpec((_H, _H), lambda i: (0, 0)),
            pl.BlockSpec((_H, _H), lambda i: (0, 0)),
        ],
        out_specs=[
            pl.BlockSpec((_BN, _H), lambda i: (i, 0)),
            pl.BlockSpec((2, _BN, _H), lambda i: (0, i, 0)),
        ],
        out_shape=[
            jax.ShapeDtypeStruct((_N, _H), jnp.float32),
            jax.ShapeDtypeStruct((2, _N, _H), jnp.float32),
        ],
    )(x, ag0, ag1, W1a, W1b, b1, W2, b2, nm, A, B)


def _node_last_body(x, a0, a1, W1a, W1b, b1, W2, b2, nm, Wo, bo, fo):
    xx = x[...]
    agg = a0[...] + a1[...]
    t = jnp.maximum(_dot(xx, W1a[...]) + _dot(agg, W1b[...]) + b1[...], 0.0)
    out = (xx + _dot(t, W2[...]) + b2[...]) * nm[...]
    fo[...] = _dot(out, Wo[...]) + bo[...]


def _node_last(x, ag0, ag1, W1a, W1b, b1, W2, b2, nm, Wo, bo):
    grid = _N // _BN
    return pl.pallas_call(
        _node_last_body,
        grid=(grid,),
        in_specs=[
            pl.BlockSpec((_BN, _H), lambda i: (i, 0)),
            pl.BlockSpec((_BN, _H), lambda i: (i, 0)),
            pl.BlockSpec((_BN, _H), lambda i: (i, 0)),
            pl.BlockSpec((_H, _H), lambda i: (0, 0)),
            pl.BlockSpec((_H, _H), lambda i: (0, 0)),
            pl.BlockSpec((1, _H), lambda i: (0, 0)),
            pl.BlockSpec((_H, _H), lambda i: (0, 0)),
            pl.BlockSpec((1, _H), lambda i: (0, 0)),
            pl.BlockSpec((_BN, 1), lambda i: (i, 0)),
            pl.BlockSpec((_H, _OUT), lambda i: (0, 0)),
            pl.BlockSpec((1, _OUT), lambda i: (0, 0)),
        ],
        out_specs=pl.BlockSpec((_BN, _OUT), lambda i: (i, 0)),
        out_shape=jax.ShapeDtypeStruct((_N, _OUT), jnp.float32),
    )(x, ag0, ag1, W1a, W1b, b1, W2, b2, nm, Wo, bo)


def kernel(h, edges, distances, node_mask, edge_mask, emb_in_W, emb_in_b,
           eW1, eb1, eW2, eb2, nW1, nb1, nW2, nb2, emb_out_W, emb_out_b):
    row = edges[0]
    col = edges[1]
    gidx = jnp.concatenate([row, col + _N])

    x, pq = _pre(h, emb_in_W, emb_in_b.reshape(1, _H),
                 eW1[0, :_H], eW1[0, _H:2 * _H])
    out = None
    for l in range(_NL):
        pq2 = pq.reshape(2 * _N, _H)
        wd = eW1[l, 2 * _H].reshape(1, _H)
        b1 = eb1[l].reshape(1, _H)
        b2 = eb2[l].reshape(1, _H)
        aggs = []
        for ph in range(2):
            g = _gather(pq2, gidx, ph)
            ef = _edge(g, distances, edge_mask, wd, b1, eW2[l], b2, ph)
            aggs.append(_scatter(ef, row, ph))
        if l < _NL - 1:
            x, pq = _node(x, aggs[0], aggs[1], nW1[l, :_H], nW1[l, _H:],
                          nb1[l].reshape(1, _H), nW2[l],
                          nb2[l].reshape(1, _H), node_mask,
                          eW1[l + 1, :_H], eW1[l + 1, _H:2 * _H])
        else:
            out = _node_last(x, aggs[0], aggs[1], nW1[l, :_H], nW1[l, _H:],
                             nb1[l].reshape(1, _H), nW2[l],
                             nb2[l].reshape(1, _H), node_mask,
                             emb_out_W, emb_out_b.reshape(1, _OUT))
    return out


# speedup vs baseline: 1.1425x; 1.1425x over previous
import functools

import jax
import jax.numpy as jnp
from jax import lax
from jax.experimental import pallas as pl
from jax.experimental.pallas import tpu as pltpu
from jax.experimental.pallas import tpu_sc as plsc

_N = 10000
_E = 320000
_EH = _E // 2
_H = 128
_NL = 3
_OUT = 10

_NC = 2
_NS = 16
_NW = _NC * _NS

_GCH = 40
_GPT = _EH // _NW
_GNCH = _GPT // _GCH

_SCH = 80
_SPT = _EH // _NS
_SNCH = _SPT // _SCH
_NH = _N // 2
_NHP = 5008
_RPT = 312
_ZCH = 52

_BE = 2000
_BN = 2000


def _dot(a, b):
    return jnp.dot(a, b, preferred_element_type=jnp.float32)


def _mesh():
    return plsc.VectorSubcoreMesh(core_axis_name="c", subcore_axis_name="s",
                                  num_cores=_NC, num_subcores=_NS)


def _gather_body(ph, pq_hbm, idx_hbm, out_hbm, idxp_v, idxq_v,
                 rows0, rows1, rows2, g0, g1, g2, w0, w1, w2):
    c = lax.axis_index("c")
    s = lax.axis_index("s")
    w = c * _NS + s
    base = ph * _EH + w * _GPT
    obase = w * _GPT
    pltpu.sync_copy(idx_hbm.at[pl.ds(base, _GPT)], idxp_v)
    pltpu.sync_copy(idx_hbm.at[pl.ds(_E + base, _GPT)], idxq_v)
    rows = (rows0, rows1, rows2)
    gs = (g0, g1, g2)
    ws = (w0, w1, w2)

    def start_p(j, b):
        off = pl.multiple_of(j * _GCH, 8)
        pltpu.async_copy(pq_hbm.at[idxp_v.at[pl.ds(off, _GCH)]], rows[b], gs[b])

    def wait_g(b):
        pltpu.make_async_copy(pq_hbm.at[pl.ds(0, _GCH)], rows[b], gs[b]).wait()

    def wait_w(b):
        pltpu.make_async_copy(rows[b], out_hbm.at[pl.ds(0, _GCH)], ws[b]).wait()

    for b in range(3):
        start_p(b, b)

    def step(jj, b, dyn):
        bp = (b + 2) % 3
        wait_g(b)
        off = pl.multiple_of(jj * _GCH, 8)
        pltpu.async_copy(pq_hbm.at[idxq_v.at[pl.ds(off, _GCH)]], rows[b],
                         gs[b], add=True)
        wait_g(b)
        dst = pl.multiple_of(obase + jj * _GCH, 8)
        pltpu.async_copy(rows[b], out_hbm.at[pl.ds(dst, _GCH)], ws[b])
        if dyn:
            @pl.when((jj >= 1) & (jj + 2 < _GNCH))
            def _():
                wait_w(bp)
                start_p(jj + 2, bp)

    def body(i, carry):
        for b in range(3):
            step(3 * i + b, b, True)
        return carry

    nfull = _GNCH - _GNCH % 3
    lax.fori_loop(0, nfull // 3, body, 0)
    for jj in range(nfull, _GNCH):
        step(jj, jj % 3, False)
    for b in range(3):
        wait_w(b)


def _gather(pq2, gidx, ph):
    f = pl.kernel(
        functools.partial(_gather_body, ph),
        out_type=jax.ShapeDtypeStruct((_EH, _H), jnp.float32),
        mesh=_mesh(),
        scratch_types=[
            pltpu.VMEM((_GPT,), jnp.int32),
            pltpu.VMEM((_GPT,), jnp.int32),
            pltpu.VMEM((_GCH, _H), jnp.float32),
            pltpu.VMEM((_GCH, _H), jnp.float32),
            pltpu.VMEM((_GCH, _H), jnp.float32),
            pltpu.SemaphoreType.DMA,
            pltpu.SemaphoreType.DMA,
            pltpu.SemaphoreType.DMA,
            pltpu.SemaphoreType.DMA,
            pltpu.SemaphoreType.DMA,
            pltpu.SemaphoreType.DMA,
        ],
    )
    return f(pq2, gidx)


def _scatter_body(ph, ef_hbm, idx_hbm, out_hbm, idx1_v, idx2_v, rows0, rows1,
                  rows2, zbuf, agg_s, g0, g1, g2, a0, a1, a2):
    c = lax.axis_index("c")
    s = lax.axis_index("s")
    base_e = s * _SPT

    def zb(i, carry):
        for k in range(8):
            zbuf[i, pl.ds(k * 16, 16)] = jnp.zeros((16,), jnp.float32)
        return carry

    lax.fori_loop(0, _ZCH, zb, 0)
    zbase = pl.multiple_of(s * _RPT, 8)
    for r in range(6):
        pltpu.sync_copy(zbuf, agg_s.at[pl.ds(zbase + r * _ZCH, _ZCH)])

    @pl.when(s == _NS - 1)
    def _():
        pltpu.sync_copy(zbuf.at[pl.ds(0, 16)], agg_s.at[pl.ds(_NS * _RPT, 16)])

    pltpu.sync_copy(idx_hbm.at[pl.ds(ph * _EH + base_e, _SPT)], idx1_v)
    lo = c * _NH

    def rp(i, carry):
        src = pl.multiple_of(i * _SCH, 16)
        for k in range(5):
            v = idx1_v[pl.ds(src + k * 16, 16)] - lo
            ok = (v >= 0) & (v < _NH)
            idx2_v[i, pl.ds(k * 16, 16)] = jnp.where(ok, v, _NH)
        return carry

    lax.fori_loop(0, _SNCH, rp, 0)
    plsc.subcore_barrier()

    rows = (rows0, rows1, rows2)
    gs = (g0, g1, g2)
    ads = (a0, a1, a2)

    def start(j, b):
        off = pl.multiple_of(base_e + j * _SCH, 8)
        pltpu.async_copy(ef_hbm.at[pl.ds(off, _SCH)], rows[b], gs[b])

    def wait_g(b):
        pltpu.make_async_copy(ef_hbm.at[pl.ds(0, _SCH)], rows[b], gs[b]).wait()

    def wait_a(b):
        pltpu.make_async_copy(rows[b], agg_s.at[pl.ds(0, _SCH)], ads[b]).wait()

    for b in range(3):
        start(b, b)

    def step(jj, b, dyn):
        bp = (b + 2) % 3
        wait_g(b)
        pltpu.async_copy(rows[b], agg_s.at[idx2_v.at[jj]], ads[b], add=True)
        if dyn:
            @pl.when((jj >= 1) & (jj + 2 < _SNCH))
            def _():
                wait_a(bp)
                start(jj + 2, bp)

    def body(i, carry):
        for b in range(3):
            step(3 * i + b, b, True)
        return carry

    nfull = _SNCH - _SNCH % 3
    lax.fori_loop(0, nfull // 3, body, 0)
    for jj in range(nfull, _SNCH):
        step(jj, jj % 3, False)
    for b in range(3):
        wait_a(b)
    plsc.subcore_barrier()
    src = pl.multiple_of(s * _RPT, 8)
    dst = pl.multiple_of(c * _NH + s * _RPT, 8)
    pltpu.sync_copy(agg_s.at[pl.ds(src, _RPT)], out_hbm.at[pl.ds(dst, _RPT)])

    @pl.when(s == _NS - 1)
    def _():
        tail = _NS * _RPT
        pltpu.sync_copy(agg_s.at[pl.ds(tail, 8)],
                        out_hbm.at[pl.ds(c * _NH + tail, 8)])


def _scatter(ef, row, ph):
    f = pl.kernel(
        functools.partial(_scatter_body, ph),
        out_type=jax.ShapeDtypeStruct((_N, _H), jnp.float32),
        mesh=_mesh(),
        scratch_types=[
            pltpu.VMEM((_SPT,), jnp.int32),
            pltpu.VMEM((_SNCH, _SCH), jnp.int32),
            pltpu.VMEM((_SCH, _H), jnp.float32),
            pltpu.VMEM((_SCH, _H), jnp.float32),
            pltpu.VMEM((_SCH, _H), jnp.float32),
            pltpu.VMEM((_ZCH, _H), jnp.float32),
            pltpu.VMEM_SHARED((_NHP, _H), jnp.float32),
            pltpu.SemaphoreType.DMA,
            pltpu.SemaphoreType.DMA,
            pltpu.SemaphoreType.DMA,
            pltpu.SemaphoreType.DMA,
            pltpu.SemaphoreType.DMA,
            pltpu.SemaphoreType.DMA,
        ],
    )
    return f(ef, row)


def _pre_body(h, Wi, bi, A, B, xo, pqo):
    x = _dot(h[...], Wi[...]) + bi[...]
    xo[...] = x
    pqo[0, :, :] = _dot(x, A[...])
    pqo[1, :, :] = _dot(x, B[...])


def _pre(h, Wi, bi, A, B):
    grid = _N // _BN
    return pl.pallas_call(
        _pre_body,
        grid=(grid,),
        in_specs=[
            pl.BlockSpec((_BN, _H), lambda i: (i, 0)),
            pl.BlockSpec((_H, _H), lambda i: (0, 0)),
            pl.BlockSpec((1, _H), lambda i: (0, 0)),
            pl.BlockSpec((_H, _H), lambda i: (0, 0)),
            pl.BlockSpec((_H, _H), lambda i: (0, 0)),
        ],
        out_specs=[
            pl.BlockSpec((_BN, _H), lambda i: (i, 0)),
            pl.BlockSpec((2, _BN, _H), lambda i: (0, i, 0)),
        ],
        out_shape=[
            jax.ShapeDtypeStruct((_N, _H), jnp.float32),
            jax.ShapeDtypeStruct((2, _N, _H), jnp.float32),
        ],
    )(h, Wi, bi, A, B)


def _edge_body(ss, d, em, wd, b1, W2, b2, o):
    h1 = jnp.maximum(ss[...] + d[...] * wd[...] + b1[...], 0.0)
    h16 = h1.astype(jnp.bfloat16)
    m = jnp.maximum(_dot(h16, W2[...].astype(jnp.bfloat16)) + b2[...], 0.0)
    o[...] = m * em[...]


def _edge(g, d, em, wd, b1, W2, b2, ph):
    grid = _EH // _BE
    nb = _EH // _BE

    def eix(i):
        return (i + ph * nb, 0)

    return pl.pallas_call(
        _edge_body,
        grid=(grid,),
        in_specs=[
            pl.BlockSpec((_BE, _H), lambda i: (i, 0)),
            pl.BlockSpec((_BE, 1), eix),
            pl.BlockSpec((_BE, 1), eix),
            pl.BlockSpec((1, _H), lambda i: (0, 0)),
            pl.BlockSpec((1, _H), lambda i: (0, 0)),
            pl.BlockSpec((_H, _H), lambda i: (0, 0)),
            pl.BlockSpec((1, _H), lambda i: (0, 0)),
        ],
        out_specs=pl.BlockSpec((_BE, _H), lambda i: (i, 0)),
        out_shape=jax.ShapeDtypeStruct((_EH, _H), jnp.float32),
    )(g, d, em, wd, b1, W2, b2)


def _node_body(x, a0, a1, W1a, W1b, b1, W2, b2, nm, A, B, xo, pqo):
    xx = x[...]
    agg = a0[...] + a1[...]
    t = jnp.maximum(_dot(xx, W1a[...]) + _dot(agg, W1b[...]) + b1[...], 0.0)
    out = (xx + _dot(t, W2[...]) + b2[...]) * nm[...]
    xo[...] = out
    pqo[0, :, :] = _dot(out, A[...])
    pqo[1, :, :] = _dot(out, B[...])


def _node(x, ag0, ag1, W1a, W1b, b1, W2, b2, nm, A, B):
    grid = _N // _BN
    return pl.pallas_call(
        _node_body,
        grid=(grid,),
        in_specs=[
            pl.BlockSpec((_BN, _H), lambda i: (i, 0)),
            pl.BlockSpec((_BN, _H), lambda i: (i, 0)),
            pl.BlockSpec((_BN, _H), lambda i: (i, 0)),
            pl.BlockSpec((_H, _H), lambda i: (0, 0)),
            pl.BlockSpec((_H, _H), lambda i: (0, 0)),
            pl.BlockSpec((1, _H), lambda i: (0, 0)),
            pl.BlockSpec((_H, _H), lambda i: (0, 0)),
            pl.BlockSpec((1, _H), lambda i: (0, 0)),
            pl.BlockSpec((_BN, 1), lambda i: (i, 0)),
            pl.BlockSpec((_H, _H), lambda i: (0, 0)),
            pl.BlockSpec((_H, _H), lambda i: (0, 0)),
        ],
        out_specs=[
            pl.BlockSpec((_BN, _H), lambda i: (i, 0)),
            pl.BlockSpec((2, _BN, _H), lambda i: (0, i, 0)),
        ],
        out_shape=[
            jax.ShapeDtypeStruct((_N, _H), jnp.float32),
            jax.ShapeDtypeStruct((2, _N, _H), jnp.float32),
        ],
    )(x, ag0, ag1, W1a, W1b, b1, W2, b2, nm, A, B)


def _node_last_body(x, a0, a1, W1a, W1b, b1, W2, b2, nm, Wo, bo, fo):
    xx = x[...]
    agg = a0[...] + a1[...]
    t = jnp.maximum(_dot(xx, W1a[...]) + _dot(agg, W1b[...]) + b1[...], 0.0)
    out = (xx + _dot(t, W2[...]) + b2[...]) * nm[...]
    fo[...] = _dot(out, Wo[...]) + bo[...]


def _node_last(x, ag0, ag1, W1a, W1b, b1, W2, b2, nm, Wo, bo):
    grid = _N // _BN
    return pl.pallas_call(
        _node_last_body,
        grid=(grid,),
        in_specs=[
            pl.BlockSpec((_BN, _H), lambda i: (i, 0)),
            pl.BlockSpec((_BN, _H), lambda i: (i, 0)),
            pl.BlockSpec((_BN, _H), lambda i: (i, 0)),
            pl.BlockSpec((_H, _H), lambda i: (0, 0)),
            pl.BlockSpec((_H, _H), lambda i: (0, 0)),
            pl.BlockSpec((1, _H), lambda i: (0, 0)),
            pl.BlockSpec((_H, _H), lambda i: (0, 0)),
            pl.BlockSpec((1, _H), lambda i: (0, 0)),
            pl.BlockSpec((_BN, 1), lambda i: (i, 0)),
            pl.BlockSpec((_H, _OUT), lambda i: (0, 0)),
            pl.BlockSpec((1, _OUT), lambda i: (0, 0)),
        ],
        out_specs=pl.BlockSpec((_BN, _OUT), lambda i: (i, 0)),
        out_shape=jax.ShapeDtypeStruct((_N, _OUT), jnp.float32),
    )(x, ag0, ag1, W1a, W1b, b1, W2, b2, nm, Wo, bo)


def kernel(h, edges, distances, node_mask, edge_mask, emb_in_W, emb_in_b,
           eW1, eb1, eW2, eb2, nW1, nb1, nW2, nb2, emb_out_W, emb_out_b):
    row = edges[0]
    col = edges[1]
    gidx = jnp.concatenate([row, col + _N])

    x, pq = _pre(h, emb_in_W, emb_in_b.reshape(1, _H),
                 eW1[0, :_H], eW1[0, _H:2 * _H])
    out = None
    for l in range(_NL):
        pq2 = pq.reshape(2 * _N, _H)
        wd = eW1[l, 2 * _H].reshape(1, _H)
        b1 = eb1[l].reshape(1, _H)
        b2 = eb2[l].reshape(1, _H)
        aggs = []
        for ph in range(2):
            g = _gather(pq2, gidx, ph)
            ef = _edge(g, distances, edge_mask, wd, b1, eW2[l], b2, ph)
            aggs.append(_scatter(ef, row, ph))
        if l < _NL - 1:
            x, pq = _node(x, aggs[0], aggs[1], nW1[l, :_H], nW1[l, _H:],
                          nb1[l].reshape(1, _H), nW2[l],
                          nb2[l].reshape(1, _H), node_mask,
                          eW1[l + 1, :_H], eW1[l + 1, _H:2 * _H])
        else:
            out = _node_last(x, aggs[0], aggs[1], nW1[l, :_H], nW1[l, _H:],
                             nb1[l].reshape(1, _H), nW2[l],
                             nb2[l].reshape(1, _H), node_mask,
                             emb_out_W, emb_out_b.reshape(1, _OUT))
    return out
